# baseline (speedup 1.0000x reference)
import functools

import jax
import jax.numpy as jnp
from jax import lax
from jax.experimental import pallas as pl
from jax.experimental.pallas import tpu as pltpu
from jax.experimental.pallas import tpu_sc as plsc

K = 512
D = 4096
NC = 2
NS = 16
NW = NC * NS

SC_ROWS = 64
TC_ROWS = K - SC_ROWS
SC_RPW = SC_ROWS // NW
BLK = 64

_mesh = plsc.VectorSubcoreMesh(core_axis_name="c", subcore_axis_name="s")


@functools.partial(
    pl.kernel,
    mesh=_mesh,
    out_type=jax.ShapeDtypeStruct((SC_ROWS, D), jnp.float32),
    scratch_types=[pltpu.VMEM((SC_RPW, D), jnp.float32)],
)
def _sc_copy(table_hbm, out_hbm, buf):
    wid = lax.axis_index("s") * NC + lax.axis_index("c")
    base = wid * SC_RPW
    pltpu.sync_copy(table_hbm.at[pl.ds(TC_ROWS + base, SC_RPW)], buf)
    pltpu.sync_copy(buf, out_hbm.at[pl.ds(base, SC_RPW)])


def _copy_body(x_ref, o_ref):
    o_ref[...] = x_ref[...]


def _stitch_body(sc_ref, _, o_ref):
    o_ref[...] = sc_ref[...]


def kernel(embedding_weight):
    sc_out = _sc_copy(embedding_weight)
    big = pl.pallas_call(
        _copy_body,
        grid=(TC_ROWS // BLK,),
        in_specs=[pl.BlockSpec((BLK, D), lambda i: (i, 0))],
        out_specs=pl.BlockSpec((BLK, D), lambda i: (i, 0)),
        out_shape=jax.ShapeDtypeStruct((K, D), jnp.float32),
    )(embedding_weight)
    out = pl.pallas_call(
        _stitch_body,
        grid=(1,),
        in_specs=[
            pl.BlockSpec((SC_ROWS, D), lambda i: (0, 0)),
            pl.BlockSpec(memory_space=pltpu.MemorySpace.HBM),
        ],
        out_specs=pl.BlockSpec((SC_ROWS, D), lambda i: (TC_ROWS // SC_ROWS, 0)),
        out_shape=jax.ShapeDtypeStruct((K, D), jnp.float32),
        input_output_aliases={1: 0},
    )(sc_out, big)
    return out


# speedup vs baseline: 2.1127x; 2.1127x over previous
import jax
import jax.numpy as jnp
from jax.experimental import pallas as pl

K = 512
D = 4096
BLK = 32


def _copy_body(x_ref, o_ref):
    o_ref[...] = x_ref[...]


def kernel(embedding_weight):
    return pl.pallas_call(
        _copy_body,
        grid=(K // BLK,),
        in_specs=[pl.BlockSpec((BLK, D), lambda i: (i, 0))],
        out_specs=pl.BlockSpec((BLK, D), lambda i: (i, 0)),
        out_shape=jax.ShapeDtypeStruct((K, D), jnp.float32),
    )(embedding_weight)


# speedup vs baseline: 3.6225x; 1.7147x over previous
import jax
import jax.numpy as jnp
from jax.experimental import pallas as pl

K = 512
D = 4096
BLK = 128


def _copy_body(x_ref, o_ref):
    o_ref[...] = x_ref[...]


def kernel(embedding_weight):
    return pl.pallas_call(
        _copy_body,
        grid=(K // BLK,),
        in_specs=[pl.BlockSpec((BLK, D), lambda i: (i, 0))],
        out_specs=pl.BlockSpec((BLK, D), lambda i: (i, 0)),
        out_shape=jax.ShapeDtypeStruct((K, D), jnp.float32),
    )(embedding_weight)


# speedup vs baseline: 4.4836x; 1.2377x over previous
import jax
import jax.numpy as jnp
from jax.experimental import pallas as pl

K = 512
D = 4096
BLK = 256


def _copy_body(x_ref, o_ref):
    o_ref[...] = x_ref[...]


def kernel(embedding_weight):
    return pl.pallas_call(
        _copy_body,
        grid=(K // BLK,),
        in_specs=[pl.BlockSpec((BLK, D), lambda i: (i, 0))],
        out_specs=pl.BlockSpec((BLK, D), lambda i: (i, 0)),
        out_shape=jax.ShapeDtypeStruct((K, D), jnp.float32),
    )(embedding_weight)
